# baseline (speedup 1.0000x reference)
import functools

import jax
import jax.numpy as jnp
from jax import lax
from jax.experimental import pallas as pl
from jax.experimental.pallas import tpu as pltpu
from jax.experimental.pallas import tpu_sc as plsc

_B, _S, _V = 64, 2048, 31
_K = 8
_NA = 20
_MASK = 28
_NW = 16
_BPW = _B // _NW
_PPW = _BPW * _K
_NH = _PPW // 16
_BB = 64


def _gumbel_body(u_ref, g_ref):
    u = u_ref[...]
    g_ref[...] = -jnp.log(-jnp.log(u + 1e-10) + 1e-10)


def _dense_body(pred_ref, sample_ref, logp_ref, flag_ref):
    x = pred_ref[...]
    m = jnp.max(x, axis=0, keepdims=True)
    sh = x - m
    lse = jnp.log(jnp.sum(jnp.exp(sh), axis=0, keepdims=True))
    logp_ref[...] = sh - lse
    flag_ref[...] = (sample_ref[...] != _MASK).astype(jnp.int32)


def _sc_body(pred_hbm, g_hbm, pes_hbm, sample_hbm, out_hbm,
             pes_v, g_v, rows_v, buf_v, sem, sem2):
    wid = lax.axis_index("s")
    b0 = wid * _BPW
    gcopy = pltpu.async_copy(g_hbm.at[wid], g_v, sem2)
    scopy = pltpu.async_copy(sample_hbm.at[pl.ds(b0, _BPW)], buf_v, sem2)
    pltpu.sync_copy(pes_hbm.at[wid], pes_v)

    lane = lax.iota(jnp.int32, 16)
    pes_h = [pes_v[pl.ds(j * 16, 16)] for j in range(_NH)]
    copies = []
    for j, pes16 in enumerate(pes_h):
        b = b0 + 2 * j + (lane >> 3)
        base = (((b >> 3) * 16 + (pes16 >> 7)) * 1024
                + (b & 7) * 128 + (pes16 & 127))
        copies += [
            pltpu.async_copy(pred_hbm.at[base + c * (_B * _S)],
                             rows_v.at[c, pl.ds(j * 16, 16)], sem)
            for c in range(_NA)
        ]
    for c in copies:
        c.wait()
    gcopy.wait()
    scopy.wait()

    for j, pes16 in enumerate(pes_h):
        best = rows_v[0, pl.ds(j * 16, 16)] + g_v[0, pl.ds(j * 16, 16)]
        besti = jnp.zeros_like(lane)
        for c in range(1, _NA):
            s = rows_v[c, pl.ds(j * 16, 16)] + g_v[c, pl.ds(j * 16, 16)]
            upd = s > best
            best = jnp.where(upd, s, best)
            besti = jnp.where(upd, jnp.int32(c), besti)

        for p in range(16):
            loc = pes16[p]
            start = pl.multiple_of(loc & -16, 16)
            row = 2 * j + (p >> 3)
            cur = buf_v[row, pl.ds(start, 16)]
            buf_v[row, pl.ds(start, 16)] = jnp.where(
                lane == (loc & 15), besti[p], cur)
    pltpu.sync_copy(buf_v, out_hbm.at[pl.ds(b0, _BPW)])


def _make_sc_sample():
    return functools.partial(
        pl.kernel,
        out_type=jax.ShapeDtypeStruct((_B, _S), jnp.int32),
        mesh=plsc.VectorSubcoreMesh(core_axis_name="c", subcore_axis_name="s",
                                    num_cores=1),
        scratch_types=[
            pltpu.VMEM((_PPW,), jnp.int32),
            pltpu.VMEM((_NA, _PPW), jnp.float32),
            pltpu.VMEM((_NA, _PPW), jnp.float32),
            pltpu.VMEM((_BPW, _S), jnp.int32),
            pltpu.SemaphoreType.DMA,
            pltpu.SemaphoreType.DMA,
        ],
    )(_sc_body)


def kernel(prediction, sample, pes_index, gumbel_u):
    pred_t = jnp.transpose(prediction, (2, 0, 1))
    pred_flat = jnp.reshape(
        jnp.transpose(jnp.reshape(prediction, (8, 8, 16, 128, _V)),
                      (4, 0, 2, 1, 3)),
        (-1,))
    u_w = jnp.reshape(
        jnp.transpose(jnp.reshape(gumbel_u, (_NW, _BPW, _K, _NA)),
                      (0, 3, 1, 2)),
        (_NW, _NA, _PPW))
    pes_w = jnp.reshape(pes_index, (_NW, _PPW))
    g_w = pl.pallas_call(
        _gumbel_body,
        out_shape=jax.ShapeDtypeStruct((_NW, _NA, _PPW), jnp.float32),
    )(u_w)
    sample_fake = _make_sc_sample()(pred_flat, g_w, pes_w, sample)
    logp_t, flag = pl.pallas_call(
        _dense_body,
        grid=(_B // _BB,),
        compiler_params=pltpu.CompilerParams(
            dimension_semantics=("parallel",)),
        in_specs=[
            pl.BlockSpec((_V, _BB, _S), lambda i: (0, i, 0)),
            pl.BlockSpec((_BB, _S), lambda i: (i, 0)),
        ],
        out_specs=[
            pl.BlockSpec((_V, _BB, _S), lambda i: (0, i, 0)),
            pl.BlockSpec((_BB, _S), lambda i: (i, 0)),
        ],
        out_shape=[
            jax.ShapeDtypeStruct((_V, _B, _S), jnp.float32),
            jax.ShapeDtypeStruct((_B, _S), jnp.int32),
        ],
    )(pred_t, sample)
    return (jnp.transpose(logp_t, (1, 2, 0)), sample_fake, flag)


# speedup vs baseline: 1.1435x; 1.1435x over previous
import functools

import jax
import jax.numpy as jnp
from jax import lax
from jax.experimental import pallas as pl
from jax.experimental.pallas import tpu as pltpu
from jax.experimental.pallas import tpu_sc as plsc

_B, _S, _V = 64, 2048, 31
_K = 8
_NA = 20
_MASK = 28
_NW = 16
_BPW = _B // _NW
_PPW = _BPW * _K
_NH = _PPW // 16
_BB = 32


def _gumbel_body(u_ref, g_ref):
    u = u_ref[...]
    g_ref[...] = -jnp.log(-jnp.log(u + 1e-10) + 1e-10)


def _dense_body(pred_ref, sample_ref, logp_ref, flag_ref):
    x = pred_ref[...]
    m = jnp.max(x, axis=0, keepdims=True)
    sh = x - m
    lse = jnp.log(jnp.sum(jnp.exp(sh), axis=0, keepdims=True))
    logp_ref[...] = sh - lse
    flag_ref[...] = (sample_ref[...] != _MASK).astype(jnp.int32)


def _sc_body(pred_hbm, g_hbm, pes_hbm, sample_hbm, out_hbm,
             pes_v, g_v, rows_v, buf_v, sem, sem2):
    wid = lax.axis_index("s")
    b0 = wid * _BPW
    gcopy = pltpu.async_copy(g_hbm.at[wid], g_v, sem2)
    scopy = pltpu.async_copy(sample_hbm.at[pl.ds(b0, _BPW)], buf_v, sem2)
    pltpu.sync_copy(pes_hbm.at[wid], pes_v)

    lane = lax.iota(jnp.int32, 16)
    pes_h = [pes_v[pl.ds(j * 16, 16)] for j in range(_NH)]
    copies = []
    for j, pes16 in enumerate(pes_h):
        b = b0 + 2 * j + (lane >> 3)
        base = (((b >> 3) * 16 + (pes16 >> 7)) * 1024
                + (b & 7) * 128 + (pes16 & 127))
        copies += [
            pltpu.async_copy(pred_hbm.at[base + c * (_B * _S)],
                             rows_v.at[c, pl.ds(j * 16, 16)], sem)
            for c in range(_NA)
        ]
    for c in copies:
        c.wait()
    gcopy.wait()
    scopy.wait()

    for j, pes16 in enumerate(pes_h):
        best = rows_v[0, pl.ds(j * 16, 16)] + g_v[0, pl.ds(j * 16, 16)]
        besti = jnp.zeros_like(lane)
        for c in range(1, _NA):
            s = rows_v[c, pl.ds(j * 16, 16)] + g_v[c, pl.ds(j * 16, 16)]
            upd = s > best
            best = jnp.where(upd, s, best)
            besti = jnp.where(upd, jnp.int32(c), besti)

        for p in range(16):
            loc = pes16[p]
            start = pl.multiple_of(loc & -16, 16)
            row = 2 * j + (p >> 3)
            cur = buf_v[row, pl.ds(start, 16)]
            buf_v[row, pl.ds(start, 16)] = jnp.where(
                lane == (loc & 15), besti[p], cur)
    pltpu.sync_copy(buf_v, out_hbm.at[pl.ds(b0, _BPW)])


def _make_sc_sample():
    return functools.partial(
        pl.kernel,
        out_type=jax.ShapeDtypeStruct((_B, _S), jnp.int32),
        mesh=plsc.VectorSubcoreMesh(core_axis_name="c", subcore_axis_name="s",
                                    num_cores=1),
        scratch_types=[
            pltpu.VMEM((_PPW,), jnp.int32),
            pltpu.VMEM((_NA, _PPW), jnp.float32),
            pltpu.VMEM((_NA, _PPW), jnp.float32),
            pltpu.VMEM((_BPW, _S), jnp.int32),
            pltpu.SemaphoreType.DMA,
            pltpu.SemaphoreType.DMA,
        ],
    )(_sc_body)


def kernel(prediction, sample, pes_index, gumbel_u):
    pred_t = jnp.transpose(prediction, (2, 0, 1))
    pred_flat = jnp.reshape(
        jnp.transpose(jnp.reshape(prediction, (8, 8, 16, 128, _V)),
                      (4, 0, 2, 1, 3)),
        (-1,))
    u_w = jnp.reshape(
        jnp.transpose(jnp.reshape(gumbel_u, (_NW, _BPW, _K, _NA)),
                      (0, 3, 1, 2)),
        (_NW, _NA, _PPW))
    pes_w = jnp.reshape(pes_index, (_NW, _PPW))
    g_w = pl.pallas_call(
        _gumbel_body,
        out_shape=jax.ShapeDtypeStruct((_NW, _NA, _PPW), jnp.float32),
    )(u_w)
    sample_fake = _make_sc_sample()(pred_flat, g_w, pes_w, sample)
    logp_t, flag = pl.pallas_call(
        _dense_body,
        grid=(_B // _BB,),
        compiler_params=pltpu.CompilerParams(
            dimension_semantics=("parallel",)),
        in_specs=[
            pl.BlockSpec((_V, _BB, _S), lambda i: (0, i, 0)),
            pl.BlockSpec((_BB, _S), lambda i: (i, 0)),
        ],
        out_specs=[
            pl.BlockSpec((_V, _BB, _S), lambda i: (0, i, 0)),
            pl.BlockSpec((_BB, _S), lambda i: (i, 0)),
        ],
        out_shape=[
            jax.ShapeDtypeStruct((_V, _B, _S), jnp.float32),
            jax.ShapeDtypeStruct((_B, _S), jnp.int32),
        ],
    )(pred_t, sample)
    return (jnp.transpose(logp_t, (1, 2, 0)), sample_fake, flag)
